# baseline (speedup 1.0000x reference)
import functools
import math

import jax
import jax.numpy as jnp
from jax import lax
from jax.experimental import pallas as pl
from jax.experimental.pallas import tpu as pltpu
from jax.experimental.pallas import tpu_sc as plsc

DELTA = 1e-5
TOPK_N = 128
NCH = 16


def _stream_block(x_ref, out_ref, cm_ref, *, blk, n):
    xb = x_ref[...]
    ch = n // NCH
    cm = jnp.concatenate(
        [jnp.max(xb[:, c * ch:(c + 1) * ch], axis=1, keepdims=True)
         for c in range(NCH)], axis=1)
    m = jnp.max(cm, axis=1, keepdims=True)
    out_ref[...] = jnp.maximum(xb - m, 0.0)
    cm_ref[...] = cm


def _make_detect(rows, n):
    try:
        info = plsc.get_sparse_core_info()
        nc, ns = info.num_cores, info.num_subcores
    except ValueError:
        nc, ns = 2, 16
    nw = nc * ns
    rpw = rows // nw
    ch = n // NCH
    mesh = plsc.VectorSubcoreMesh(core_axis_name="c", subcore_axis_name="s")

    @functools.partial(
        pl.kernel, mesh=mesh,
        out_type=jax.ShapeDtypeStruct((rows,), jnp.int32),
        scratch_types=[pltpu.VMEM((rpw, NCH), jnp.float32),
                       pltpu.VMEM((ch,), jnp.float32),
                       pltpu.VMEM((rpw,), jnp.int32)],
        compiler_params=pltpu.CompilerParams(needs_layout_passes=False),
    )
    def detect(cm_hbm, xc_hbm, flags_hbm, cmt_v, chunk_v, fl_v):
        wid = lax.axis_index("s") * nc + lax.axis_index("c")
        base = wid * rpw
        pltpu.sync_copy(cm_hbm.at[pl.ds(base, rpw)], cmt_v)

        lane = lax.iota(jnp.int32, 16)

        def row_body(g, r, fl):
            cm = cmt_v[g * 16 + r]
            m = jnp.max(cm)
            thr = m - DELTA
            strictc = (cm >= thr) & (cm < m)
            nstrict = jnp.max(plsc.all_reduce_population_count(strictc))
            maxmask = cm == m
            nmax = jnp.max(plsc.all_reduce_population_count(maxmask))
            amax = jnp.max(plsc.all_reduce_ffs(maxmask))
            pltpu.sync_copy(xc_hbm.at[(base + g * 16 + r) * NCH + amax],
                            chunk_v)

            def scan_body(i, cnt):
                v = chunk_v[pl.ds(i * 16, 16)]
                s = (v >= thr) & (v < m)
                return cnt + jnp.max(plsc.all_reduce_population_count(s))

            cnt = lax.fori_loop(0, ch // 16, scan_body, jnp.int32(0))
            heavy = (nstrict > 0) | (nmax > 1) | (cnt > 0)
            return jnp.where(lane == r, jnp.zeros((16,), jnp.int32)
                             + heavy.astype(jnp.int32), fl)

        for g in range(rpw // 16):
            fl = lax.fori_loop(0, 16, functools.partial(row_body, g),
                               jnp.zeros((16,), jnp.int32))
            fl_v[g * 16:(g + 1) * 16] = fl
        pltpu.sync_copy(fl_v, flags_hbm.at[pl.ds(base, rpw)])

    return detect


def _exact_tau(d, cur_ref, blk):
    cur_ref[...] = jnp.maximum(d, -DELTA)
    lane = jax.lax.broadcasted_iota(jnp.int32, (blk, TOPK_N), 1)

    def body(_, carry):
        acc, filled = carry
        cur = cur_ref[...]
        v = jnp.max(cur, axis=1, keepdims=True)
        eqm = cur == v
        q = jnp.sum(eqm.astype(jnp.int32), axis=1, keepdims=True)
        cur_ref[...] = jnp.where(eqm, -3.0 * DELTA, cur)
        emit = (lane >= filled) & (lane < filled + q)
        acc = jnp.where(emit, v, acc)
        return acc, filled + q

    acc0 = jnp.zeros((blk, TOPK_N), jnp.float32)
    fill0 = jnp.zeros((blk, 1), jnp.int32)
    acc, _ = jax.lax.fori_loop(0, TOPK_N, body, (acc0, fill0))

    tri = (jax.lax.broadcasted_iota(jnp.int32, (TOPK_N, TOPK_N), 0)
           <= jax.lax.broadcasted_iota(jnp.int32, (TOPK_N, TOPK_N), 1)
           ).astype(jnp.float32)
    c = jax.lax.dot_general(acc, tri, (((1,), (0,)), ((), ())),
                            preferred_element_type=jnp.float32)
    ranks = (lane + 1).astype(jnp.float32)
    ratio = c / (acc + 1e-8)
    maskk = ranks <= ratio
    supp = jnp.clip(jnp.sum(maskk.astype(jnp.int32), axis=1,
                            keepdims=True), 1, None)
    csel = jnp.sum(jnp.where(lane == supp - 1, c, 0.0), axis=1,
                   keepdims=True)
    return csel / (supp.astype(jnp.float32) + 1e-8)


def _fixup(flv_ref, fls_ref, x_any, out0_any, out_any,
           row_v, cur_v, sem_in, sem_out, *, rows, n):
    del out0_any
    nflag = jnp.sum(flv_ref[...])

    @pl.when(nflag > 0)
    def _any_heavy():
        def row_body(r, _):
            @pl.when(fls_ref[r // 256, r % 256] > 0)
            def _fix():
                cin = pltpu.make_async_copy(x_any.at[r], row_v.at[0], sem_in)
                cin.start()
                cin.wait()
                xr = row_v[...]
                m = jnp.max(xr, axis=1, keepdims=True)
                d = xr - m
                tau = _exact_tau(d, cur_v, 1)
                row_v[...] = jnp.maximum(d - tau, 0.0)
                cout = pltpu.make_async_copy(row_v.at[0], out_any.at[r],
                                             sem_out)
                cout.start()
                cout.wait()
            return 0

        lax.fori_loop(0, rows, row_body, 0)


@jax.jit
def kernel(x):
    b, h, n = x.shape
    rows = b * h
    blk = math.gcd(16, rows)
    xf = x.reshape(rows, n)

    out0, cm = pl.pallas_call(
        functools.partial(_stream_block, blk=blk, n=n),
        grid=(rows // blk,),
        in_specs=[pl.BlockSpec((blk, n), lambda i: (i, 0))],
        out_specs=[pl.BlockSpec((blk, n), lambda i: (i, 0)),
                   pl.BlockSpec((blk, NCH), lambda i: (i, 0))],
        out_shape=[jax.ShapeDtypeStruct((rows, n), jnp.float32),
                   jax.ShapeDtypeStruct((rows, NCH), jnp.float32)],
        compiler_params=pltpu.CompilerParams(
            dimension_semantics=("parallel",)),
    )(xf)

    flags = _make_detect(rows, n)(cm, xf.reshape(rows * NCH, n // NCH))

    flags2 = flags.reshape(rows // 256, 256)
    out = pl.pallas_call(
        functools.partial(_fixup, rows=rows, n=n),
        in_specs=[pl.BlockSpec(memory_space=pltpu.MemorySpace.VMEM),
                  pl.BlockSpec(memory_space=pltpu.MemorySpace.SMEM),
                  pl.BlockSpec(memory_space=pl.ANY),
                  pl.BlockSpec(memory_space=pl.ANY)],
        out_specs=pl.BlockSpec(memory_space=pl.ANY),
        out_shape=jax.ShapeDtypeStruct((rows, n), jnp.float32),
        scratch_shapes=[pltpu.VMEM((1, n), jnp.float32),
                        pltpu.VMEM((1, n), jnp.float32),
                        pltpu.SemaphoreType.DMA,
                        pltpu.SemaphoreType.DMA],
        input_output_aliases={3: 0},
    )(flags2, flags2, xf, out0)
    return out.reshape(b, h, n)


# speedup vs baseline: 1.0813x; 1.0813x over previous
import functools
import math

import jax
import jax.numpy as jnp
from jax import lax
from jax.experimental import pallas as pl
from jax.experimental.pallas import tpu as pltpu
from jax.experimental.pallas import tpu_sc as plsc

DELTA = 1e-5
TOPK_N = 128
NCH = 16


def _stream_block(x_ref, out_ref, cm_ref, *, blk, n):
    xb = x_ref[...]
    ch = n // NCH
    cm = jnp.concatenate(
        [jnp.max(xb[:, c * ch:(c + 1) * ch], axis=1, keepdims=True)
         for c in range(NCH)], axis=1)
    m = jnp.max(cm, axis=1, keepdims=True)
    out_ref[...] = jnp.maximum(xb - m, 0.0)
    cm_ref[...] = cm


def _make_detect(rows, n):
    try:
        info = plsc.get_sparse_core_info()
        nc, ns = info.num_cores, info.num_subcores
    except ValueError:
        nc, ns = 2, 16
    nw = nc * ns
    rpw = rows // nw
    ch = n // NCH
    mesh = plsc.VectorSubcoreMesh(core_axis_name="c", subcore_axis_name="s")

    @functools.partial(
        pl.kernel, mesh=mesh,
        out_type=jax.ShapeDtypeStruct((rows,), jnp.int32),
        scratch_types=[pltpu.VMEM((rpw, NCH), jnp.float32),
                       pltpu.VMEM((rpw,), jnp.int32),
                       pltpu.VMEM((rpw,), jnp.int32),
                       pltpu.VMEM((16, ch), jnp.float32),
                       pltpu.VMEM((16, ch), jnp.float32),
                       pltpu.SemaphoreType.DMA,
                       pltpu.SemaphoreType.DMA],
        compiler_params=pltpu.CompilerParams(needs_layout_passes=False),
    )
    def detect(cm_hbm, xc_hbm, flags_hbm, cmt_v, idx_v, fl_v,
               buf0, buf1, sem0, sem1):
        wid = lax.axis_index("s") * nc + lax.axis_index("c")
        base = wid * rpw
        pltpu.sync_copy(cm_hbm.at[pl.ds(base, rpw)], cmt_v)
        lane = lax.iota(jnp.int32, 16)
        ngrp = rpw // 16

        def pre_body(g, r, carry):
            fl, idx = carry
            cm = cmt_v[g * 16 + r]
            m = jnp.max(cm)
            strictc = (cm >= m - DELTA) & (cm < m)
            nstrict = jnp.max(plsc.all_reduce_population_count(strictc))
            maxmask = cm == m
            nmax = jnp.max(plsc.all_reduce_population_count(maxmask))
            amax = jnp.max(plsc.all_reduce_ffs(maxmask))
            pre = ((nstrict > 0) | (nmax > 1)).astype(jnp.int32)
            sel = lane == r
            fl = jnp.where(sel, jnp.zeros((16,), jnp.int32) + pre, fl)
            idx = jnp.where(sel, jnp.zeros((16,), jnp.int32)
                            + (base + g * 16 + r) * NCH + amax, idx)
            return fl, idx

        for g in range(ngrp):
            fl, idx = lax.fori_loop(
                0, 16, functools.partial(pre_body, g),
                (jnp.zeros((16,), jnp.int32), jnp.zeros((16,), jnp.int32)))
            fl_v[g * 16:(g + 1) * 16] = fl
            idx_v[g * 16:(g + 1) * 16] = idx

        bufs = (buf0, buf1)
        sems = (sem0, sem1)
        cps = [pltpu.async_copy(
            xc_hbm.at[idx_v.at[pl.ds(0, 16)]], bufs[0], sems[0])]
        for g in range(ngrp):
            cps[g].wait()
            if g + 1 < ngrp:
                cps.append(pltpu.async_copy(
                    xc_hbm.at[idx_v.at[pl.ds((g + 1) * 16, 16)]],
                    bufs[(g + 1) % 2], sems[(g + 1) % 2]))
            buf = bufs[g % 2]

            def scan_row(r, fl):
                cm = cmt_v[g * 16 + r]
                m = jnp.max(cm)
                neg = m - 1.0

                def scan_body(i, acc):
                    for u in range(8):
                        v = buf[r, pl.ds((i * 8 + u) * 16, 16)]
                        acc = jnp.maximum(acc, jnp.where(v < m, v, neg))
                    return acc

                acc = lax.fori_loop(0, ch // 128, scan_body,
                                    jnp.zeros((16,), jnp.float32) + neg)
                strict = (jnp.max(acc) >= m - DELTA).astype(jnp.int32)
                return jnp.where(lane == r, jnp.maximum(fl, strict), fl)

            fl = lax.fori_loop(0, 16, scan_row, jnp.zeros((16,), jnp.int32))
            fl_v[g * 16:(g + 1) * 16] = jnp.maximum(
                fl_v[g * 16:(g + 1) * 16], fl)
        pltpu.sync_copy(fl_v, flags_hbm.at[pl.ds(base, rpw)])

    return detect


def _exact_tau(d, cur_ref, blk):
    cur_ref[...] = jnp.maximum(d, -DELTA)
    lane = jax.lax.broadcasted_iota(jnp.int32, (blk, TOPK_N), 1)

    def body(_, carry):
        acc, filled = carry
        cur = cur_ref[...]
        v = jnp.max(cur, axis=1, keepdims=True)
        eqm = cur == v
        q = jnp.sum(eqm.astype(jnp.int32), axis=1, keepdims=True)
        cur_ref[...] = jnp.where(eqm, -3.0 * DELTA, cur)
        emit = (lane >= filled) & (lane < filled + q)
        acc = jnp.where(emit, v, acc)
        return acc, filled + q

    acc0 = jnp.zeros((blk, TOPK_N), jnp.float32)
    fill0 = jnp.zeros((blk, 1), jnp.int32)
    acc, _ = jax.lax.fori_loop(0, TOPK_N, body, (acc0, fill0))

    tri = (jax.lax.broadcasted_iota(jnp.int32, (TOPK_N, TOPK_N), 0)
           <= jax.lax.broadcasted_iota(jnp.int32, (TOPK_N, TOPK_N), 1)
           ).astype(jnp.float32)
    c = jax.lax.dot_general(acc, tri, (((1,), (0,)), ((), ())),
                            preferred_element_type=jnp.float32)
    ranks = (lane + 1).astype(jnp.float32)
    ratio = c / (acc + 1e-8)
    maskk = ranks <= ratio
    supp = jnp.clip(jnp.sum(maskk.astype(jnp.int32), axis=1,
                            keepdims=True), 1, None)
    csel = jnp.sum(jnp.where(lane == supp - 1, c, 0.0), axis=1,
                   keepdims=True)
    return csel / (supp.astype(jnp.float32) + 1e-8)


def _fixup(flv_ref, fls_ref, x_any, out0_any, out_any,
           row_v, cur_v, sem_in, sem_out, *, rows, n):
    del out0_any
    nflag = jnp.sum(flv_ref[...])

    @pl.when(nflag > 0)
    def _any_heavy():
        def row_body(r, _):
            @pl.when(fls_ref[r // 256, r % 256] > 0)
            def _fix():
                cin = pltpu.make_async_copy(x_any.at[r], row_v.at[0], sem_in)
                cin.start()
                cin.wait()
                xr = row_v[...]
                m = jnp.max(xr, axis=1, keepdims=True)
                d = xr - m
                tau = _exact_tau(d, cur_v, 1)
                row_v[...] = jnp.maximum(d - tau, 0.0)
                cout = pltpu.make_async_copy(row_v.at[0], out_any.at[r],
                                             sem_out)
                cout.start()
                cout.wait()
            return 0

        lax.fori_loop(0, rows, row_body, 0)


@jax.jit
def kernel(x):
    b, h, n = x.shape
    rows = b * h
    blk = math.gcd(16, rows)
    xf = x.reshape(rows, n)

    out0, cm = pl.pallas_call(
        functools.partial(_stream_block, blk=blk, n=n),
        grid=(rows // blk,),
        in_specs=[pl.BlockSpec((blk, n), lambda i: (i, 0))],
        out_specs=[pl.BlockSpec((blk, n), lambda i: (i, 0)),
                   pl.BlockSpec((blk, NCH), lambda i: (i, 0))],
        out_shape=[jax.ShapeDtypeStruct((rows, n), jnp.float32),
                   jax.ShapeDtypeStruct((rows, NCH), jnp.float32)],
        compiler_params=pltpu.CompilerParams(
            dimension_semantics=("parallel",)),
    )(xf)

    flags = _make_detect(rows, n)(cm, xf.reshape(rows * NCH, n // NCH))

    flags2 = flags.reshape(rows // 256, 256)
    out = pl.pallas_call(
        functools.partial(_fixup, rows=rows, n=n),
        in_specs=[pl.BlockSpec(memory_space=pltpu.MemorySpace.VMEM),
                  pl.BlockSpec(memory_space=pltpu.MemorySpace.SMEM),
                  pl.BlockSpec(memory_space=pl.ANY),
                  pl.BlockSpec(memory_space=pl.ANY)],
        out_specs=pl.BlockSpec(memory_space=pl.ANY),
        out_shape=jax.ShapeDtypeStruct((rows, n), jnp.float32),
        scratch_shapes=[pltpu.VMEM((1, n), jnp.float32),
                        pltpu.VMEM((1, n), jnp.float32),
                        pltpu.SemaphoreType.DMA,
                        pltpu.SemaphoreType.DMA],
        input_output_aliases={3: 0},
    )(flags2, flags2, xf, out0)
    return out.reshape(b, h, n)


# speedup vs baseline: 4.3592x; 4.0313x over previous
import functools
import math

import jax
import jax.numpy as jnp
from jax.experimental import pallas as pl
from jax.experimental.pallas import tpu as pltpu

DELTA = 1e-5
TOPK_N = 128


def _soft_threshold_block(x_ref, out_ref, cur_ref, *, blk, n):
    xb = x_ref[...]
    m = jnp.max(xb, axis=1, keepdims=True)
    d = xb - m
    out_ref[...] = jnp.maximum(d, 0.0)
    v2 = jnp.max(jnp.where(d < 0.0, d, -1.0))

    @pl.when(v2 >= -DELTA)
    def _heavy():
        cur_ref[...] = jnp.maximum(d, -DELTA)
        lane = jax.lax.broadcasted_iota(jnp.int32, (blk, TOPK_N), 1)

        def body(_, carry):
            acc, filled = carry
            cur = cur_ref[...]
            v = jnp.max(cur, axis=1, keepdims=True)
            eqm = cur == v
            q = jnp.sum(eqm.astype(jnp.int32), axis=1, keepdims=True)
            cur_ref[...] = jnp.where(eqm, -3.0 * DELTA, cur)
            emit = (lane >= filled) & (lane < filled + q)
            acc = jnp.where(emit, v, acc)
            return acc, filled + q

        acc0 = jnp.zeros((blk, TOPK_N), jnp.float32)
        fill0 = jnp.zeros((blk, 1), jnp.int32)
        acc, _ = jax.lax.fori_loop(0, TOPK_N, body, (acc0, fill0))

        tri = (jax.lax.broadcasted_iota(jnp.int32, (TOPK_N, TOPK_N), 0)
               <= jax.lax.broadcasted_iota(jnp.int32, (TOPK_N, TOPK_N), 1)
               ).astype(jnp.float32)
        c = jax.lax.dot_general(acc, tri, (((1,), (0,)), ((), ())),
                                preferred_element_type=jnp.float32)
        ranks = (lane + 1).astype(jnp.float32)
        ratio = c / (acc + 1e-8)
        maskk = ranks <= ratio
        supp = jnp.clip(jnp.sum(maskk.astype(jnp.int32), axis=1,
                                keepdims=True), 1, None)
        csel = jnp.sum(jnp.where(lane == supp - 1, c, 0.0), axis=1,
                       keepdims=True)
        tau = csel / (supp.astype(jnp.float32) + 1e-8)
        out_ref[...] = jnp.maximum(d - tau, 0.0)


@jax.jit
def kernel(x):
    b, h, n = x.shape
    rows = b * h
    blk = math.gcd(32, rows)
    xf = x.reshape(rows, n)
    body = functools.partial(_soft_threshold_block, blk=blk, n=n)
    out = pl.pallas_call(
        body,
        grid=(rows // blk,),
        in_specs=[pl.BlockSpec((blk, n), lambda i: (i, 0))],
        out_specs=pl.BlockSpec((blk, n), lambda i: (i, 0)),
        out_shape=jax.ShapeDtypeStruct((rows, n), jnp.float32),
        scratch_shapes=[pltpu.VMEM((blk, n), jnp.float32)],
        compiler_params=pltpu.CompilerParams(
            dimension_semantics=("parallel",)),
    )(xf)
    return out.reshape(b, h, n)


# speedup vs baseline: 4.8048x; 1.1022x over previous
import functools
import math

import jax
import jax.numpy as jnp
from jax.experimental import pallas as pl
from jax.experimental.pallas import tpu as pltpu

DELTA = 1e-5
TOPK_N = 128


def _soft_threshold_block(x_ref, out_ref, cur_ref, *, blk, n):
    xb = x_ref[...]
    m = jnp.max(xb, axis=1, keepdims=True)
    d = xb - m
    out_ref[...] = jnp.maximum(d, 0.0)
    v2 = jnp.max(jnp.where(d < 0.0, d, -1.0))

    @pl.when(v2 >= -DELTA)
    def _heavy():
        cur_ref[...] = jnp.maximum(d, -DELTA)
        lane = jax.lax.broadcasted_iota(jnp.int32, (blk, TOPK_N), 1)

        def body(_, carry):
            acc, filled = carry
            cur = cur_ref[...]
            v = jnp.max(cur, axis=1, keepdims=True)
            eqm = cur == v
            q = jnp.sum(eqm.astype(jnp.int32), axis=1, keepdims=True)
            cur_ref[...] = jnp.where(eqm, -3.0 * DELTA, cur)
            emit = (lane >= filled) & (lane < filled + q)
            acc = jnp.where(emit, v, acc)
            return acc, filled + q

        acc0 = jnp.zeros((blk, TOPK_N), jnp.float32)
        fill0 = jnp.zeros((blk, 1), jnp.int32)
        acc, _ = jax.lax.fori_loop(0, TOPK_N, body, (acc0, fill0))

        tri = (jax.lax.broadcasted_iota(jnp.int32, (TOPK_N, TOPK_N), 0)
               <= jax.lax.broadcasted_iota(jnp.int32, (TOPK_N, TOPK_N), 1)
               ).astype(jnp.float32)
        c = jax.lax.dot_general(acc, tri, (((1,), (0,)), ((), ())),
                                preferred_element_type=jnp.float32)
        ranks = (lane + 1).astype(jnp.float32)
        ratio = c / (acc + 1e-8)
        maskk = ranks <= ratio
        supp = jnp.clip(jnp.sum(maskk.astype(jnp.int32), axis=1,
                                keepdims=True), 1, None)
        csel = jnp.sum(jnp.where(lane == supp - 1, c, 0.0), axis=1,
                       keepdims=True)
        tau = csel / (supp.astype(jnp.float32) + 1e-8)
        out_ref[...] = jnp.maximum(d - tau, 0.0)


@jax.jit
def kernel(x):
    b, h, n = x.shape
    rows = b * h
    blk = math.gcd(64, rows)
    xf = x.reshape(rows, n)
    body = functools.partial(_soft_threshold_block, blk=blk, n=n)
    out = pl.pallas_call(
        body,
        grid=(rows // blk,),
        in_specs=[pl.BlockSpec((blk, n), lambda i: (i, 0))],
        out_specs=pl.BlockSpec((blk, n), lambda i: (i, 0)),
        out_shape=jax.ShapeDtypeStruct((rows, n), jnp.float32),
        scratch_shapes=[pltpu.VMEM((blk, n), jnp.float32)],
        compiler_params=pltpu.CompilerParams(
            dimension_semantics=("parallel",)),
    )(xf)
    return out.reshape(b, h, n)
